# baseline (speedup 1.0000x reference)
import functools

import jax
import jax.numpy as jnp
from jax import lax
from jax.experimental import pallas as pl
from jax.experimental.pallas import tpu as pltpu
from jax.experimental.pallas import tpu_sc as plsc

B, T, V = 1024, 50, 1000
N = B * T
NC, NS, L = 2, 16, 16
NW = NC * NS
PER = N // NW
CHUNK = 80
NCHUNK = PER // CHUNK
NGROUP = PER // L

_mesh = plsc.VectorSubcoreMesh(core_axis_name="c", subcore_axis_name="s")


@functools.partial(
    pl.kernel,
    out_type=[
        jax.ShapeDtypeStruct((NW, L), jnp.float32),
        jax.ShapeDtypeStruct((NW, L), jnp.float32),
    ],
    mesh=_mesh,
    scratch_types=[
        pltpu.VMEM((PER,), jnp.int32),
        pltpu.VMEM((PER,), jnp.float32),
        pltpu.VMEM((PER,), jnp.float32),
        pltpu.VMEM((L,), jnp.float32),
        pltpu.VMEM((L,), jnp.float32),
        pltpu.SemaphoreType.DMA,
    ],
)
def _sc_gather(flat_hbm, tgt_hbm, mask_hbm, prod_out, mask_out,
               idx_v, vals_v, mask_v, acc_v, macc_v, sem):
    wid = lax.axis_index("s") * NC + lax.axis_index("c")
    base = wid * PER

    pltpu.sync_copy(tgt_hbm.at[pl.ds(base, PER)], idx_v)
    pltpu.sync_copy(mask_hbm.at[pl.ds(base, PER)], mask_v)

    lane = lax.iota(jnp.int32, L)
    for j in range(NGROUP):
        sl = pl.ds(j * L, L)
        row = (base + j * L) + lane
        idx_v[sl] = row * V + idx_v[sl]

    copies = []
    for c in range(NCHUNK):
        sl = pl.ds(c * CHUNK, CHUNK)
        copies.append(pltpu.async_copy(flat_hbm.at[idx_v.at[sl]],
                                       vals_v.at[sl], sem))
    for cp in copies:
        cp.wait()

    acc = jnp.zeros((L,), jnp.float32)
    macc = jnp.zeros((L,), jnp.float32)
    for j in range(NGROUP):
        sl = pl.ds(j * L, L)
        m = mask_v[sl]
        acc = acc + vals_v[sl] * m
        macc = macc + m
    acc_v[...] = acc
    macc_v[...] = macc
    pltpu.sync_copy(acc_v, prod_out.at[wid])
    pltpu.sync_copy(macc_v, mask_out.at[wid])


def _finish_body(p_ref, m_ref, o_ref):
    s = -jnp.sum(p_ref[...]) / jnp.sum(m_ref[...])
    o_ref[...] = jnp.full((1, 1), s, jnp.float32)


_finish = pl.pallas_call(
    _finish_body,
    out_shape=jax.ShapeDtypeStruct((1, 1), jnp.float32),
)


def kernel(input, target, mask):
    flat = input.reshape(-1)
    tgt = target.reshape(-1).astype(jnp.int32)
    m = mask.reshape(-1)
    prod_p, mask_p = _sc_gather(flat, tgt, m)
    return _finish(prod_p, mask_p)[0, 0]


# speedup vs baseline: 11.9763x; 11.9763x over previous
import dataclasses
import functools

import jax
import jax.numpy as jnp
from jax import lax
from jax.experimental import pallas as pl
from jax.experimental.pallas import tpu as pltpu
from jax.experimental.pallas import tpu_sc as plsc

B, T, V = 1024, 50, 1000
N = B * T
NC, NS, L = 2, 16, 16
NW = NC * NS
LANES = 128
CPW = 32
PER = T * CPW
CHUNK = 80
HALF = PER // 2
NCHUNK = HALF // CHUNK

_mesh = plsc.VectorSubcoreMesh(core_axis_name="c", subcore_axis_name="s")

_cp = pltpu.CompilerParams()
if "needs_layout_passes" in pltpu.CompilerParams.__dataclass_fields__:
    _cp = dataclasses.replace(_cp, needs_layout_passes=False)


@functools.partial(
    pl.kernel,
    out_type=[
        jax.ShapeDtypeStruct((NW, L), jnp.float32),
        jax.ShapeDtypeStruct((NW, L), jnp.float32),
    ],
    mesh=_mesh,
    scratch_types=[
        pltpu.VMEM((PER,), jnp.int32),
        pltpu.VMEM((HALF, LANES), jnp.float32),
        pltpu.VMEM((PER,), jnp.float32),
        pltpu.VMEM((L,), jnp.float32),
        pltpu.VMEM((L,), jnp.float32),
        pltpu.SemaphoreType.DMA,
    ],
    compiler_params=_cp,
)
def _sc_gather(x2_hbm, rows_hbm, mask_hbm, prod_out, mask_out,
               idx_v, buf_v, mask_v, acc_v, macc_v, sem):
    wid = lax.axis_index("s") * NC + lax.axis_index("c")
    base = wid * PER

    pltpu.sync_copy(rows_hbm.at[pl.ds(base, PER)], idx_v)
    pltpu.sync_copy(mask_hbm.at[pl.ds(base, PER)], mask_v)

    col_tile = pl.multiple_of((wid // 4) * LANES, LANES)
    c_base = (wid % 4) * CPW

    lane = lax.iota(jnp.int32, L)
    acc = jnp.zeros((L,), jnp.float32)
    macc = jnp.zeros((L,), jnp.float32)
    for half in range(2):
        copies = []
        for c in range(NCHUNK):
            sl = pl.ds(half * HALF + c * CHUNK, CHUNK)
            dsl = pl.ds(c * CHUNK, CHUNK)
            copies.append(pltpu.async_copy(
                x2_hbm.at[idx_v.at[sl], pl.ds(col_tile, LANES)],
                buf_v.at[dsl], sem))
        for cp in copies:
            cp.wait()
        for j in range(HALF // L):
            cix = c_base + (j % 2) * L + lane
            vals = plsc.load_gather(buf_v, [j * L + lane, cix])
            m = mask_v[pl.ds(half * HALF + j * L, L)]
            acc = acc + vals * m
            macc = macc + m
    acc_v[...] = acc
    macc_v[...] = macc
    pltpu.sync_copy(acc_v, prod_out.at[wid])
    pltpu.sync_copy(macc_v, mask_out.at[wid])


def _finish_body(p_ref, m_ref, o_ref):
    s = -jnp.sum(p_ref[...]) / jnp.sum(m_ref[...])
    o_ref[...] = jnp.full((1, 1), s, jnp.float32)


_finish = pl.pallas_call(
    _finish_body,
    out_shape=jax.ShapeDtypeStruct((1, 1), jnp.float32),
)


def kernel(input, target, mask):
    x2 = input.transpose(1, 2, 0).reshape(T * V, B)
    tT = target.T.astype(jnp.int32)
    rows = jnp.arange(T, dtype=jnp.int32)[:, None] * V + tT
    rows_g = rows.reshape(T, 8, 4, CPW).transpose(1, 2, 0, 3).reshape(-1)
    mask_g = mask.T.reshape(T, 8, 4, CPW).transpose(1, 2, 0, 3).reshape(-1)
    prod_p, mask_p = _sc_gather(x2, rows_g, mask_g)
    return _finish(prod_p, mask_p)[0, 0]


# speedup vs baseline: 14.1145x; 1.1785x over previous
import dataclasses
import functools

import jax
import jax.numpy as jnp
from jax import lax
from jax.experimental import pallas as pl
from jax.experimental.pallas import tpu as pltpu
from jax.experimental.pallas import tpu_sc as plsc

B, T, V = 1024, 50, 1000
N = B * T
NC, NS, L = 2, 16, 16
NW = NC * NS
LANES = 128
CPW = 32
PER = T * CPW
CHUNK = 80
HALF = PER // 2
NCHUNK = HALF // CHUNK

_mesh = plsc.VectorSubcoreMesh(core_axis_name="c", subcore_axis_name="s")

_cp = pltpu.CompilerParams()
if "needs_layout_passes" in pltpu.CompilerParams.__dataclass_fields__:
    _cp = dataclasses.replace(_cp, needs_layout_passes=False)


@functools.partial(
    pl.kernel,
    out_type=[
        jax.ShapeDtypeStruct((NW, L), jnp.float32),
        jax.ShapeDtypeStruct((NW, L), jnp.float32),
    ],
    mesh=_mesh,
    scratch_types=[
        pltpu.VMEM((PER,), jnp.int32),
        pltpu.VMEM((HALF, LANES), jnp.float32),
        pltpu.VMEM((PER,), jnp.float32),
        pltpu.VMEM((L,), jnp.float32),
        pltpu.VMEM((L,), jnp.float32),
        pltpu.SemaphoreType.DMA,
    ],
    compiler_params=_cp,
)
def _sc_gather(x2_hbm, rows_hbm, mask_hbm, prod_out, mask_out,
               idx_v, buf_v, mask_v, acc_v, macc_v, sem):
    wid = lax.axis_index("s") * NC + lax.axis_index("c")
    base = wid * PER

    pltpu.sync_copy(rows_hbm.at[pl.ds(base, PER)], idx_v)
    pltpu.sync_copy(mask_hbm.at[pl.ds(base, PER)], mask_v)

    col_tile = pl.multiple_of((wid // 4) * LANES, LANES)
    c_base = (wid % 4) * CPW

    lane = lax.iota(jnp.int32, L)
    acc_v[...] = jnp.zeros((L,), jnp.float32)
    macc_v[...] = jnp.zeros((L,), jnp.float32)
    for half in range(2):
        def _chunk_copy(c):
            sl = pl.ds(pl.multiple_of(half * HALF + c * CHUNK, 8), CHUNK)
            dsl = pl.ds(pl.multiple_of(c * CHUNK, 8), CHUNK)
            return pltpu.make_async_copy(
                x2_hbm.at[idx_v.at[sl], pl.ds(col_tile, LANES)],
                buf_v.at[dsl], sem)

        @pl.loop(0, NCHUNK)
        def _(c):
            _chunk_copy(c).start()

        @pl.loop(0, NCHUNK)
        def _(c):
            _chunk_copy(c).wait()

        @pl.loop(0, HALF // L)
        def _(j):
            cix = c_base + lax.rem(j, 2) * L + lane
            vals = plsc.load_gather(buf_v, [j * L + lane, cix])
            m = mask_v[pl.ds(pl.multiple_of(half * HALF + j * L, 8), L)]
            acc_v[...] += vals * m
            macc_v[...] += m
    pltpu.sync_copy(acc_v, prod_out.at[wid])
    pltpu.sync_copy(macc_v, mask_out.at[wid])


def _finish_body(p_ref, m_ref, o_ref):
    s = -jnp.sum(p_ref[...]) / jnp.sum(m_ref[...])
    o_ref[...] = jnp.full((1, 1), s, jnp.float32)


_finish = pl.pallas_call(
    _finish_body,
    out_shape=jax.ShapeDtypeStruct((1, 1), jnp.float32),
)


def kernel(input, target, mask):
    x2 = input.transpose(1, 2, 0).reshape(T * V, B)
    tT = target.T.astype(jnp.int32)
    rows = jnp.arange(T, dtype=jnp.int32)[:, None] * V + tT
    rows_g = rows.reshape(T, 8, 4, CPW).transpose(1, 2, 0, 3).reshape(-1)
    mask_g = mask.T.reshape(T, 8, 4, CPW).transpose(1, 2, 0, 3).reshape(-1)
    prod_p, mask_p = _sc_gather(x2, rows_g, mask_g)
    return _finish(prod_p, mask_p)[0, 0]


# speedup vs baseline: 19.4180x; 1.3757x over previous
import functools

import jax
import jax.numpy as jnp
from jax import lax
from jax.experimental import pallas as pl
from jax.experimental.pallas import tpu as pltpu
from jax.experimental.pallas import tpu_sc as plsc

B, T, V = 1024, 50, 1000
N = B * T
NC, NS, L = 2, 16, 16
NW = NC * NS
PER = N // NW
CHUNK = 80
NCHUNK = PER // CHUNK

_mesh = plsc.VectorSubcoreMesh(core_axis_name="c", subcore_axis_name="s")


@functools.partial(
    pl.kernel,
    out_type=[
        jax.ShapeDtypeStruct((NW, L), jnp.float32),
        jax.ShapeDtypeStruct((NW, L), jnp.float32),
    ],
    mesh=_mesh,
    scratch_types=[
        pltpu.VMEM((PER,), jnp.int32),
        pltpu.VMEM((PER,), jnp.float32),
        pltpu.VMEM((PER,), jnp.float32),
        pltpu.VMEM((L,), jnp.float32),
        pltpu.VMEM((L,), jnp.float32),
        pltpu.SemaphoreType.DMA,
    ],
)
def _sc_gather(flat_hbm, rows_hbm, mask_hbm, prod_out, mask_out,
               idx_v, vals_v, mask_v, acc_v, macc_v, sem):
    wid = lax.axis_index("s") * NC + lax.axis_index("c")
    base = wid * PER

    pltpu.sync_copy(rows_hbm.at[pl.ds(base, PER)], idx_v)
    pltpu.sync_copy(mask_hbm.at[pl.ds(base, PER)], mask_v)

    def _chunk_copy(c):
        sl = pl.ds(pl.multiple_of(c * CHUNK, 8), CHUNK)
        return pltpu.make_async_copy(
            flat_hbm.at[idx_v.at[sl]], vals_v.at[sl], sem)

    @pl.loop(0, NCHUNK)
    def _(c):
        _chunk_copy(c).start()

    @pl.loop(0, NCHUNK)
    def _(c):
        _chunk_copy(c).wait()

    acc_v[...] = jnp.zeros((L,), jnp.float32)
    macc_v[...] = jnp.zeros((L,), jnp.float32)

    @pl.loop(0, PER // L)
    def _(j):
        sl = pl.ds(pl.multiple_of(j * L, 8), L)
        m = mask_v[sl]
        acc_v[...] += vals_v[sl] * m
        macc_v[...] += m

    pltpu.sync_copy(acc_v, prod_out.at[wid])
    pltpu.sync_copy(macc_v, mask_out.at[wid])


def _finish_body(p_ref, m_ref, o_ref):
    s = -jnp.sum(p_ref[...]) / jnp.sum(m_ref[...])
    o_ref[...] = jnp.full((1, 1), s, jnp.float32)


_finish = pl.pallas_call(
    _finish_body,
    out_shape=jax.ShapeDtypeStruct((1, 1), jnp.float32),
)


def kernel(input, target, mask):
    x1 = (input.transpose(1, 2, 0)
          .reshape(T, V // 8, 8, B // 128, 128)
          .transpose(0, 1, 3, 2, 4)
          .reshape(-1))
    tT = target.T.astype(jnp.int32)
    t_col = jnp.arange(T, dtype=jnp.int32)[:, None]
    b_row = jnp.arange(B, dtype=jnp.int32)[None, :]
    rows = ((t_col * (V // 8) + tT // 8) * 64 + (b_row // 128) * 8
            + tT % 8) * 128 + b_row % 128
    prod_p, mask_p = _sc_gather(x1, rows.reshape(-1), mask.T.reshape(-1))
    return _finish(prod_p, mask_p)[0, 0]
